# baseline (speedup 1.0000x reference)
import jax
import jax.numpy as jnp
from jax import lax
from jax.experimental import pallas as pl
from jax.experimental.pallas import tpu as pltpu
from jax.experimental.pallas import tpu_sc as plsc

NC = 2
NS = 16
L = 16
NW = NC * NS

BATCH = 16384
D = 32
LINE = 128
RPL = LINE // D
BPW = BATCH // NW
CHUNK = 128
NCHUNK = BPW // CHUNK


def _mf_body(utid_hbm, itid_hbm, usub_hbm, isub_hbm, uemb_hbm, iemb_hbm,
             out_hbm, utid, itid, usub, isub, ubuf, ibuf, outv, sem):
    wid = lax.axis_index("s") * NC + lax.axis_index("c")

    pltpu.sync_copy(utid_hbm.at[wid], utid)
    pltpu.sync_copy(itid_hbm.at[wid], itid)
    pltpu.sync_copy(usub_hbm.at[wid], usub)
    pltpu.sync_copy(isub_hbm.at[wid], isub)

    def fire(c):
        b = c % 2
        cu = pltpu.async_copy(uemb_hbm.at[utid.at[c]], ubuf.at[b], sem)
        ci = pltpu.async_copy(iemb_hbm.at[itid.at[c]], ibuf.at[b], sem)
        return cu, ci

    last_lane = lax.broadcasted_iota(jnp.int32, (L,), 0) == (L - 1)
    pending = fire(0)

    for c in range(NCHUNK):
        pending[0].wait()
        pending[1].wait()
        if c + 1 < NCHUNK:
            pending = fire(c + 1)
        b = c % 2

        def group_body(g, _, c=c, b=b):
            subv_u = usub[c, pl.ds(g * L, L)] * D
            subv_i = isub[c, pl.ds(g * L, L)] * D
            for j in range(L):
                l = g * L + j
                su = subv_u[j]
                sv = subv_i[j]
                acc = (ubuf[b, l, pl.ds(su, L)] * ibuf[b, l, pl.ds(sv, L)]
                       + ubuf[b, l, pl.ds(su + L, L)]
                       * ibuf[b, l, pl.ds(sv + L, L)])
                total = plsc.cumsum(acc)
                plsc.store_scatter(
                    outv, [jnp.full((L,), c * CHUNK + l, jnp.int32)], total,
                    mask=last_lane)
            return 0

        lax.fori_loop(0, CHUNK // L, group_body, 0)

    pltpu.sync_copy(outv, out_hbm.at[pl.ds(wid * BPW, BPW)])


@jax.jit
def kernel(x, user_embedding, item_embedding):
    uids = x[:, 0].astype(jnp.int32)
    iids = x[:, 1].astype(jnp.int32)
    utid = (uids // RPL).reshape(NW, NCHUNK, CHUNK)
    itid = (iids // RPL).reshape(NW, NCHUNK, CHUNK)
    usub = (uids % RPL).reshape(NW, NCHUNK, CHUNK)
    isub = (iids % RPL).reshape(NW, NCHUNK, CHUNK)
    uemb = user_embedding.reshape(-1, LINE)
    iemb = item_embedding.reshape(-1, LINE)
    mesh = plsc.VectorSubcoreMesh(core_axis_name="c", subcore_axis_name="s")
    run = pl.kernel(
        _mf_body,
        out_type=jax.ShapeDtypeStruct((BATCH,), jnp.float32),
        mesh=mesh,
        compiler_params=pltpu.CompilerParams(
            needs_layout_passes=False, use_tc_tiling_on_sc=False),
        scratch_types=[
            pltpu.VMEM((NCHUNK, CHUNK), jnp.int32),
            pltpu.VMEM((NCHUNK, CHUNK), jnp.int32),
            pltpu.VMEM((NCHUNK, CHUNK), jnp.int32),
            pltpu.VMEM((NCHUNK, CHUNK), jnp.int32),
            pltpu.VMEM((2, CHUNK, LINE), jnp.float32),
            pltpu.VMEM((2, CHUNK, LINE), jnp.float32),
            pltpu.VMEM((BPW,), jnp.float32),
            pltpu.SemaphoreType.DMA,
        ],
    )
    return run(utid, itid, usub, isub, uemb, iemb)


# speedup vs baseline: 1.0046x; 1.0046x over previous
import jax
import jax.numpy as jnp
from jax import lax
from jax.experimental import pallas as pl
from jax.experimental.pallas import tpu as pltpu
from jax.experimental.pallas import tpu_sc as plsc

NC = 2
NS = 16
L = 16
NW = NC * NS

BATCH = 16384
D = 32
BPW = BATCH // NW
CHUNK = 128
NCHUNK = BPW // CHUNK


def _mf_body(uids_hbm, iids_hbm, uemb_hbm, iemb_hbm, out_hbm,
             uidx, iidx, urows, irows, outv, sem):
    wid = lax.axis_index("s") * NC + lax.axis_index("c")

    pltpu.sync_copy(uids_hbm.at[wid], uidx)
    pltpu.sync_copy(iids_hbm.at[wid], iidx)

    copies = []
    for c in range(NCHUNK):
        copies.append(pltpu.async_copy(
            uemb_hbm.at[uidx.at[c]],
            urows.at[pl.ds(c * CHUNK, CHUNK), :], sem))
        copies.append(pltpu.async_copy(
            iemb_hbm.at[iidx.at[c]],
            irows.at[pl.ds(c * CHUNK, CHUNK), :], sem))
    for cp in copies:
        cp.wait()

    last_lane = lax.broadcasted_iota(jnp.int32, (L,), 0) == (L - 1)

    def row_body(r, _):
        acc = (urows[r, pl.ds(0, L)] * irows[r, pl.ds(0, L)]
               + urows[r, pl.ds(L, L)] * irows[r, pl.ds(L, L)])
        total = plsc.cumsum(acc)
        plsc.store_scatter(outv, [jnp.full((L,), r, jnp.int32)], total,
                           mask=last_lane)
        return 0

    lax.fori_loop(0, BPW, row_body, 0)

    pltpu.sync_copy(outv, out_hbm.at[pl.ds(wid * BPW, BPW)])


@jax.jit
def kernel(x, user_embedding, item_embedding):
    uids = x[:, 0].astype(jnp.int32).reshape(NW, NCHUNK, CHUNK)
    iids = x[:, 1].astype(jnp.int32).reshape(NW, NCHUNK, CHUNK)
    mesh = plsc.VectorSubcoreMesh(core_axis_name="c", subcore_axis_name="s")
    run = pl.kernel(
        _mf_body,
        out_type=jax.ShapeDtypeStruct((BATCH,), jnp.float32),
        mesh=mesh,
        compiler_params=pltpu.CompilerParams(
            needs_layout_passes=False, use_tc_tiling_on_sc=False),
        scratch_types=[
            pltpu.VMEM((NCHUNK, CHUNK), jnp.int32),
            pltpu.VMEM((NCHUNK, CHUNK), jnp.int32),
            pltpu.VMEM((BPW, D), jnp.float32),
            pltpu.VMEM((BPW, D), jnp.float32),
            pltpu.VMEM((BPW,), jnp.float32),
            pltpu.SemaphoreType.DMA,
        ],
    )
    return run(uids, iids, user_embedding, item_embedding)
